# baseline (speedup 1.0000x reference)
import jax
import jax.numpy as jnp
from jax.experimental import pallas as pl
from jax.experimental.pallas import tpu as pltpu

C = 128
PAD = 128


def _wavenet_body(x_ref, wi_ref, bi_ref, wb_ref, bb_ref, wf_ref, bf_ref,
                  out_ref, xs, x0s, y1s, *, L):
    zh = jnp.zeros((C, PAD), jnp.bfloat16)
    for s in (xs, x0s, y1s):
        s[:, 0:PAD] = zh
        s[:, PAD + L:] = zh

    xs[:, PAD:PAD + L] = x_ref[0].astype(jnp.bfloat16)

    def taps(src, d):
        return jnp.concatenate(
            [src[:, PAD - d:PAD - d + L],
             src[:, PAD:PAD + L],
             src[:, PAD + d:PAD + d + L]], axis=0)

    z = jnp.dot(wi_ref[...], taps(xs, 1),
                preferred_element_type=jnp.float32) + bi_ref[...]
    x0s[:, PAD:PAD + L] = z.astype(jnp.bfloat16)

    def gate(h):
        u = jnp.exp(-jnp.maximum(h, -30.0))
        return (1.0 - u) / (1.0 + u * u)

    for li in range(4):
        d = 1 << li

        def block(src, idx):
            zz = jnp.dot(wb_ref[idx], taps(src, d),
                         preferred_element_type=jnp.float32) + bb_ref[idx]
            return gate(zz[:C, :]) + zz[C:, :]

        y1s[:, PAD:PAD + L] = block(x0s, li).astype(jnp.bfloat16)
        y2 = block(y1s, 4 + li).astype(jnp.bfloat16)
        contrib = jnp.dot(wf_ref[li], y2, preferred_element_type=jnp.float32)
        if li == 0:
            out_ref[0] = contrib + bf_ref[...]
        else:
            out_ref[0] = out_ref[0] + contrib


def _fuse_block(cw, cb, rw, rb):
    w = jnp.zeros((3 * C, 2 * C), jnp.float32)
    w = w.at[:, :C].set(cw.reshape(3 * C, C))
    w = w.at[C:2 * C, C:].set(rw)
    b = jnp.concatenate([cb, rb], axis=-1)
    return w.T, b.T


def kernel(x, iw, ib, fw, fb,
           s0l0_cw, s0l0_cb, s0l0_rw, s0l0_rb,
           s0l1_cw, s0l1_cb, s0l1_rw, s0l1_rb,
           s0l2_cw, s0l2_cb, s0l2_rw, s0l2_rb,
           s0l3_cw, s0l3_cb, s0l3_rw, s0l3_rb,
           s1l0_cw, s1l0_cb, s1l0_rw, s1l0_rb, s1l0_sw, s1l0_sb,
           s1l1_cw, s1l1_cb, s1l1_rw, s1l1_rb, s1l1_sw, s1l1_sb,
           s1l2_cw, s1l2_cb, s1l2_rw, s1l2_rb, s1l2_sw, s1l2_sb,
           s1l3_cw, s1l3_cb, s1l3_rw, s1l3_rb, s1l3_sw, s1l3_sb):
    B, _, L = x.shape
    Lp = L + 2 * PAD

    blocks = [
        (s0l0_cw, s0l0_cb, s0l0_rw, s0l0_rb),
        (s0l1_cw, s0l1_cb, s0l1_rw, s0l1_rb),
        (s0l2_cw, s0l2_cb, s0l2_rw, s0l2_rb),
        (s0l3_cw, s0l3_cb, s0l3_rw, s0l3_rb),
        (s1l0_cw, s1l0_cb, s1l0_rw, s1l0_rb),
        (s1l1_cw, s1l1_cb, s1l1_rw, s1l1_rb),
        (s1l2_cw, s1l2_cb, s1l2_rw, s1l2_rb),
        (s1l3_cw, s1l3_cb, s1l3_rw, s1l3_rb),
    ]
    wb_list, bb_list = [], []
    for cw, cb, rw, rb in blocks:
        wt, bt = _fuse_block(cw, cb, rw, rb)
        wb_list.append(wt)
        bb_list.append(bt)
    wb = jnp.stack(wb_list).astype(jnp.bfloat16)
    bb = jnp.stack(bb_list)

    wt_init = iw.reshape(3 * C, C).T.astype(jnp.bfloat16)
    b_init = ib.T

    wf_chain = jnp.stack([(sw @ fw).T for sw in
                          (s1l0_sw, s1l1_sw, s1l2_sw, s1l3_sw)]
                         ).astype(jnp.bfloat16)
    bsf = ((s1l0_sb + s1l1_sb + s1l2_sb + s1l3_sb) @ fw + fb).T

    import functools
    body = functools.partial(_wavenet_body, L=L)

    return pl.pallas_call(
        body,
        out_shape=jax.ShapeDtypeStruct((B, C, L), jnp.float32),
        grid=(B,),
        in_specs=[
            pl.BlockSpec((1, C, L), lambda b: (b, 0, 0)),
            pl.BlockSpec((C, 3 * C), lambda b: (0, 0)),
            pl.BlockSpec((C, 1), lambda b: (0, 0)),
            pl.BlockSpec((8, 2 * C, 3 * C), lambda b: (0, 0, 0)),
            pl.BlockSpec((8, 2 * C, 1), lambda b: (0, 0, 0)),
            pl.BlockSpec((4, C, C), lambda b: (0, 0, 0)),
            pl.BlockSpec((C, 1), lambda b: (0, 0)),
        ],
        out_specs=pl.BlockSpec((1, C, L), lambda b: (b, 0, 0)),
        scratch_shapes=[
            pltpu.VMEM((C, Lp), jnp.bfloat16),
            pltpu.VMEM((C, Lp), jnp.bfloat16),
            pltpu.VMEM((C, Lp), jnp.bfloat16),
        ],
        compiler_params=pltpu.CompilerParams(
            dimension_semantics=("parallel",)),
    )(x, wt_init, b_init, wb, bb, wf_chain, bsf)


# speedup vs baseline: 1.3998x; 1.3998x over previous
import functools

import jax
import jax.numpy as jnp
from jax.experimental import pallas as pl
from jax.experimental.pallas import tpu as pltpu

C = 128
PAD = 16


def _wavenet_body(x_ref, wi_ref, bi_ref, wb_ref, bb_ref, wf_ref, bf_ref,
                  out_ref, xs, x0s, y1s, *, L):
    zh = jnp.zeros((PAD, C), jnp.bfloat16)
    for s in (xs, x0s, y1s):
        s[0:PAD, :] = zh
        s[PAD + L:, :] = zh

    xs[PAD:PAD + L, :] = x_ref[0]

    def taps(src, d):
        return jnp.concatenate(
            [src[PAD - d:PAD - d + L, :],
             src[PAD:PAD + L, :],
             src[PAD + d:PAD + d + L, :]], axis=1)

    z = jnp.dot(taps(xs, 1), wi_ref[...],
                preferred_element_type=jnp.float32) + bi_ref[...]
    x0s[PAD:PAD + L, :] = z.astype(jnp.bfloat16)

    def gate(h):
        u = jnp.exp(-jnp.maximum(h, -30.0))
        return (1.0 - u) / (1.0 + u * u)

    for li in range(4):
        d = 1 << li

        def block(src, idx):
            zz = jnp.dot(taps(src, d), wb_ref[idx],
                         preferred_element_type=jnp.float32) + bb_ref[idx]
            return gate(zz[:, :C]) + zz[:, C:]

        y1s[PAD:PAD + L, :] = block(x0s, li).astype(jnp.bfloat16)
        y2 = block(y1s, 4 + li).astype(jnp.bfloat16)
        contrib = jnp.dot(y2, wf_ref[li], preferred_element_type=jnp.float32)
        if li == 0:
            out_ref[0] = contrib + bf_ref[...]
        else:
            out_ref[0] = out_ref[0] + contrib


def _fuse_block(cw, cb, rw, rb):
    w = jnp.zeros((3 * C, 2 * C), jnp.float32)
    w = w.at[:, :C].set(cw.reshape(3 * C, C))
    w = w.at[C:2 * C, C:].set(rw)
    b = jnp.concatenate([cb, rb], axis=-1)
    return w, b


def kernel(x, iw, ib, fw, fb,
           s0l0_cw, s0l0_cb, s0l0_rw, s0l0_rb,
           s0l1_cw, s0l1_cb, s0l1_rw, s0l1_rb,
           s0l2_cw, s0l2_cb, s0l2_rw, s0l2_rb,
           s0l3_cw, s0l3_cb, s0l3_rw, s0l3_rb,
           s1l0_cw, s1l0_cb, s1l0_rw, s1l0_rb, s1l0_sw, s1l0_sb,
           s1l1_cw, s1l1_cb, s1l1_rw, s1l1_rb, s1l1_sw, s1l1_sb,
           s1l2_cw, s1l2_cb, s1l2_rw, s1l2_rb, s1l2_sw, s1l2_sb,
           s1l3_cw, s1l3_cb, s1l3_rw, s1l3_rb, s1l3_sw, s1l3_sb):
    B, _, L = x.shape
    Lp = L + 2 * PAD

    blocks = [
        (s0l0_cw, s0l0_cb, s0l0_rw, s0l0_rb),
        (s0l1_cw, s0l1_cb, s0l1_rw, s0l1_rb),
        (s0l2_cw, s0l2_cb, s0l2_rw, s0l2_rb),
        (s0l3_cw, s0l3_cb, s0l3_rw, s0l3_rb),
        (s1l0_cw, s1l0_cb, s1l0_rw, s1l0_rb),
        (s1l1_cw, s1l1_cb, s1l1_rw, s1l1_rb),
        (s1l2_cw, s1l2_cb, s1l2_rw, s1l2_rb),
        (s1l3_cw, s1l3_cb, s1l3_rw, s1l3_rb),
    ]
    wb_list, bb_list = [], []
    for cw, cb, rw, rb in blocks:
        wt, bt = _fuse_block(cw, cb, rw, rb)
        wb_list.append(wt)
        bb_list.append(bt)
    wb = jnp.stack(wb_list).astype(jnp.bfloat16)
    bb = jnp.stack(bb_list)

    wi = iw.reshape(3 * C, C).astype(jnp.bfloat16)

    wf_chain = jnp.stack([sw @ fw for sw in
                          (s1l0_sw, s1l1_sw, s1l2_sw, s1l3_sw)]
                         ).astype(jnp.bfloat16)
    bsf = (s1l0_sb + s1l1_sb + s1l2_sb + s1l3_sb) @ fw + fb

    body = functools.partial(_wavenet_body, L=L)

    xt = jnp.transpose(x, (0, 2, 1)).astype(jnp.bfloat16)

    out = pl.pallas_call(
        body,
        out_shape=jax.ShapeDtypeStruct((B, L, C), jnp.float32),
        grid=(B,),
        in_specs=[
            pl.BlockSpec((1, L, C), lambda b: (b, 0, 0)),
            pl.BlockSpec((3 * C, C), lambda b: (0, 0)),
            pl.BlockSpec((1, C), lambda b: (0, 0)),
            pl.BlockSpec((8, 3 * C, 2 * C), lambda b: (0, 0, 0)),
            pl.BlockSpec((8, 1, 2 * C), lambda b: (0, 0, 0)),
            pl.BlockSpec((4, C, C), lambda b: (0, 0, 0)),
            pl.BlockSpec((1, C), lambda b: (0, 0)),
        ],
        out_specs=pl.BlockSpec((1, L, C), lambda b: (b, 0, 0)),
        scratch_shapes=[
            pltpu.VMEM((Lp, C), jnp.bfloat16),
            pltpu.VMEM((Lp, C), jnp.bfloat16),
            pltpu.VMEM((Lp, C), jnp.bfloat16),
        ],
        compiler_params=pltpu.CompilerParams(
            dimension_semantics=("parallel",)),
    )(xt, wi, ib, wb, bb, wf_chain, bsf)

    return jnp.transpose(out, (0, 2, 1))


# speedup vs baseline: 1.4022x; 1.0018x over previous
import functools

import jax
import jax.numpy as jnp
from jax.experimental import pallas as pl
from jax.experimental.pallas import tpu as pltpu

C = 128
PAD = 16


def _wavenet_body(x_ref, wi_ref, bi_ref, wb_ref, bb_ref, wf_ref, bf_ref,
                  out_ref, xs, x0s, y1a, y1b, y1c, y1d, *, L):
    y1s_all = (y1a, y1b, y1c, y1d)
    zh = jnp.zeros((PAD, C), jnp.bfloat16)
    for s in (xs, x0s) + y1s_all:
        s[0:PAD, :] = zh
        s[PAD + L:, :] = zh

    xs[PAD:PAD + L, :] = x_ref[0]

    def taps(src, d):
        return jnp.concatenate(
            [src[PAD - d:PAD - d + L, :],
             src[PAD:PAD + L, :],
             src[PAD + d:PAD + d + L, :]], axis=1)

    z = jnp.dot(taps(xs, 1), wi_ref[...],
                preferred_element_type=jnp.float32) + bi_ref[...]
    x0s[PAD:PAD + L, :] = z.astype(jnp.bfloat16)

    def gate(h):
        u = jnp.exp(-jnp.maximum(h, -30.0))
        return (1.0 - u) / (1.0 + u * u)

    for li in range(4):
        d = 1 << li
        y1s = y1s_all[li]

        def block(src, idx):
            zz = jnp.dot(taps(src, d), wb_ref[idx],
                         preferred_element_type=jnp.float32) + bb_ref[idx]
            return gate(zz[:, :C]) + zz[:, C:]

        y1s[PAD:PAD + L, :] = block(x0s, li).astype(jnp.bfloat16)
        y2 = block(y1s, 4 + li).astype(jnp.bfloat16)
        contrib = jnp.dot(y2, wf_ref[li], preferred_element_type=jnp.float32)
        if li == 0:
            out_ref[0] = contrib + bf_ref[...]
        else:
            out_ref[0] = out_ref[0] + contrib


def _fuse_block(cw, cb, rw, rb):
    w = jnp.zeros((3 * C, 2 * C), jnp.float32)
    w = w.at[:, :C].set(cw.reshape(3 * C, C))
    w = w.at[C:2 * C, C:].set(rw)
    b = jnp.concatenate([cb, rb], axis=-1)
    return w, b


def kernel(x, iw, ib, fw, fb,
           s0l0_cw, s0l0_cb, s0l0_rw, s0l0_rb,
           s0l1_cw, s0l1_cb, s0l1_rw, s0l1_rb,
           s0l2_cw, s0l2_cb, s0l2_rw, s0l2_rb,
           s0l3_cw, s0l3_cb, s0l3_rw, s0l3_rb,
           s1l0_cw, s1l0_cb, s1l0_rw, s1l0_rb, s1l0_sw, s1l0_sb,
           s1l1_cw, s1l1_cb, s1l1_rw, s1l1_rb, s1l1_sw, s1l1_sb,
           s1l2_cw, s1l2_cb, s1l2_rw, s1l2_rb, s1l2_sw, s1l2_sb,
           s1l3_cw, s1l3_cb, s1l3_rw, s1l3_rb, s1l3_sw, s1l3_sb):
    B, _, L = x.shape
    Lp = L + 2 * PAD

    blocks = [
        (s0l0_cw, s0l0_cb, s0l0_rw, s0l0_rb),
        (s0l1_cw, s0l1_cb, s0l1_rw, s0l1_rb),
        (s0l2_cw, s0l2_cb, s0l2_rw, s0l2_rb),
        (s0l3_cw, s0l3_cb, s0l3_rw, s0l3_rb),
        (s1l0_cw, s1l0_cb, s1l0_rw, s1l0_rb),
        (s1l1_cw, s1l1_cb, s1l1_rw, s1l1_rb),
        (s1l2_cw, s1l2_cb, s1l2_rw, s1l2_rb),
        (s1l3_cw, s1l3_cb, s1l3_rw, s1l3_rb),
    ]
    wb_list, bb_list = [], []
    for cw, cb, rw, rb in blocks:
        wt, bt = _fuse_block(cw, cb, rw, rb)
        wb_list.append(wt)
        bb_list.append(bt)
    wb = jnp.stack(wb_list).astype(jnp.bfloat16)
    bb = jnp.stack(bb_list)

    wi = iw.reshape(3 * C, C).astype(jnp.bfloat16)

    wf_chain = jnp.stack([sw @ fw for sw in
                          (s1l0_sw, s1l1_sw, s1l2_sw, s1l3_sw)]
                         ).astype(jnp.bfloat16)
    bsf = (s1l0_sb + s1l1_sb + s1l2_sb + s1l3_sb) @ fw + fb

    body = functools.partial(_wavenet_body, L=L)

    xt = jnp.transpose(x, (0, 2, 1)).astype(jnp.bfloat16)

    out = pl.pallas_call(
        body,
        out_shape=jax.ShapeDtypeStruct((B, L, C), jnp.float32),
        grid=(B,),
        in_specs=[
            pl.BlockSpec((1, L, C), lambda b: (b, 0, 0)),
            pl.BlockSpec((3 * C, C), lambda b: (0, 0)),
            pl.BlockSpec((1, C), lambda b: (0, 0)),
            pl.BlockSpec((8, 3 * C, 2 * C), lambda b: (0, 0, 0)),
            pl.BlockSpec((8, 1, 2 * C), lambda b: (0, 0, 0)),
            pl.BlockSpec((4, C, C), lambda b: (0, 0, 0)),
            pl.BlockSpec((1, C), lambda b: (0, 0)),
        ],
        out_specs=pl.BlockSpec((1, L, C), lambda b: (b, 0, 0)),
        scratch_shapes=[
            pltpu.VMEM((Lp, C), jnp.bfloat16),
            pltpu.VMEM((Lp, C), jnp.bfloat16),
            pltpu.VMEM((Lp, C), jnp.bfloat16),
            pltpu.VMEM((Lp, C), jnp.bfloat16),
            pltpu.VMEM((Lp, C), jnp.bfloat16),
            pltpu.VMEM((Lp, C), jnp.bfloat16),
        ],
        compiler_params=pltpu.CompilerParams(
            dimension_semantics=("parallel",)),
    )(xt, wi, ib, wb, bb, wf_chain, bsf)

    return jnp.transpose(out, (0, 2, 1))


# speedup vs baseline: 1.6513x; 1.1776x over previous
import functools

import jax
import jax.numpy as jnp
from jax.experimental import pallas as pl
from jax.experimental.pallas import tpu as pltpu

C = 128
PAD = 16


def _wavenet_body(x_ref, wi_ref, bi_ref, wb_ref, bb_ref, wf_ref, bf_ref,
                  out_ref, xs, x0s, y1a, y1b, y1c, y1d, *, L):
    y1s_all = (y1a, y1b, y1c, y1d)
    zh = jnp.zeros((PAD, C), jnp.bfloat16)
    for s in (xs, x0s) + y1s_all:
        s[0:PAD, :] = zh
        s[PAD + L:, :] = zh

    xs[PAD:PAD + L, :] = jnp.transpose(x_ref[0].astype(jnp.bfloat16))

    def taps(src, d):
        return jnp.concatenate(
            [src[PAD - d:PAD - d + L, :],
             src[PAD:PAD + L, :],
             src[PAD + d:PAD + d + L, :]], axis=1)

    z = jnp.dot(taps(xs, 1), wi_ref[...],
                preferred_element_type=jnp.float32) + bi_ref[...]
    x0s[PAD:PAD + L, :] = z.astype(jnp.bfloat16)

    def gate(h):
        u = jnp.exp(-jnp.maximum(h, -30.0))
        return (1.0 - u) / (1.0 + u * u)

    for li in range(4):
        d = 1 << li
        y1s = y1s_all[li]

        def block(src, idx):
            zz = jnp.dot(taps(src, d), wb_ref[idx],
                         preferred_element_type=jnp.float32) + bb_ref[idx]
            return gate(zz[:, :C]) + zz[:, C:]

        y1s[PAD:PAD + L, :] = block(x0s, li).astype(jnp.bfloat16)
        y2 = block(y1s, 4 + li).astype(jnp.bfloat16)
        contrib = jnp.dot(y2, wf_ref[li], preferred_element_type=jnp.float32)
        if li == 0:
            acc = contrib + bf_ref[...]
        else:
            acc = acc + contrib
    out_ref[0] = jnp.transpose(acc)


def _fuse_block(cw, cb, rw, rb):
    w = jnp.zeros((3 * C, 2 * C), jnp.float32)
    w = w.at[:, :C].set(cw.reshape(3 * C, C))
    w = w.at[C:2 * C, C:].set(rw)
    b = jnp.concatenate([cb, rb], axis=-1)
    return w, b


def kernel(x, iw, ib, fw, fb,
           s0l0_cw, s0l0_cb, s0l0_rw, s0l0_rb,
           s0l1_cw, s0l1_cb, s0l1_rw, s0l1_rb,
           s0l2_cw, s0l2_cb, s0l2_rw, s0l2_rb,
           s0l3_cw, s0l3_cb, s0l3_rw, s0l3_rb,
           s1l0_cw, s1l0_cb, s1l0_rw, s1l0_rb, s1l0_sw, s1l0_sb,
           s1l1_cw, s1l1_cb, s1l1_rw, s1l1_rb, s1l1_sw, s1l1_sb,
           s1l2_cw, s1l2_cb, s1l2_rw, s1l2_rb, s1l2_sw, s1l2_sb,
           s1l3_cw, s1l3_cb, s1l3_rw, s1l3_rb, s1l3_sw, s1l3_sb):
    B, _, L = x.shape
    Lp = L + 2 * PAD

    blocks = [
        (s0l0_cw, s0l0_cb, s0l0_rw, s0l0_rb),
        (s0l1_cw, s0l1_cb, s0l1_rw, s0l1_rb),
        (s0l2_cw, s0l2_cb, s0l2_rw, s0l2_rb),
        (s0l3_cw, s0l3_cb, s0l3_rw, s0l3_rb),
        (s1l0_cw, s1l0_cb, s1l0_rw, s1l0_rb),
        (s1l1_cw, s1l1_cb, s1l1_rw, s1l1_rb),
        (s1l2_cw, s1l2_cb, s1l2_rw, s1l2_rb),
        (s1l3_cw, s1l3_cb, s1l3_rw, s1l3_rb),
    ]
    wb_list, bb_list = [], []
    for cw, cb, rw, rb in blocks:
        wt, bt = _fuse_block(cw, cb, rw, rb)
        wb_list.append(wt)
        bb_list.append(bt)
    wb = jnp.stack(wb_list).astype(jnp.bfloat16)
    bb = jnp.stack(bb_list)

    wi = iw.reshape(3 * C, C).astype(jnp.bfloat16)

    wf_chain = jnp.stack([sw @ fw for sw in
                          (s1l0_sw, s1l1_sw, s1l2_sw, s1l3_sw)]
                         ).astype(jnp.bfloat16)
    bsf = (s1l0_sb + s1l1_sb + s1l2_sb + s1l3_sb) @ fw + fb

    body = functools.partial(_wavenet_body, L=L)

    return pl.pallas_call(
        body,
        out_shape=jax.ShapeDtypeStruct((B, C, L), jnp.float32),
        grid=(B,),
        in_specs=[
            pl.BlockSpec((1, C, L), lambda b: (b, 0, 0)),
            pl.BlockSpec((3 * C, C), lambda b: (0, 0)),
            pl.BlockSpec((1, C), lambda b: (0, 0)),
            pl.BlockSpec((8, 3 * C, 2 * C), lambda b: (0, 0, 0)),
            pl.BlockSpec((8, 1, 2 * C), lambda b: (0, 0, 0)),
            pl.BlockSpec((4, C, C), lambda b: (0, 0, 0)),
            pl.BlockSpec((1, C), lambda b: (0, 0)),
        ],
        out_specs=pl.BlockSpec((1, C, L), lambda b: (b, 0, 0)),
        scratch_shapes=[
            pltpu.VMEM((Lp, C), jnp.bfloat16),
            pltpu.VMEM((Lp, C), jnp.bfloat16),
            pltpu.VMEM((Lp, C), jnp.bfloat16),
            pltpu.VMEM((Lp, C), jnp.bfloat16),
            pltpu.VMEM((Lp, C), jnp.bfloat16),
            pltpu.VMEM((Lp, C), jnp.bfloat16),
        ],
        compiler_params=pltpu.CompilerParams(
            dimension_semantics=("parallel",)),
    )(x, wi, ib, wb, bb, wf_chain, bsf)


# speedup vs baseline: 1.6814x; 1.0182x over previous
import functools

import jax
import jax.numpy as jnp
from jax.experimental import pallas as pl
from jax.experimental.pallas import tpu as pltpu

C = 128
PAD = 16


def _wavenet_body(x_ref, wi_ref, bi_ref, wb_ref, bb_ref, wf_ref, bf_ref,
                  out_ref, xs, x0s, y1a, y1b, y1c, y1d, *, L):
    y1s_all = (y1a, y1b, y1c, y1d)
    zh = jnp.zeros((PAD, C), jnp.bfloat16)
    for s in (xs, x0s) + y1s_all:
        s[0:PAD, :] = zh
        s[PAD + L:, :] = zh

    xs[PAD:PAD + L, :] = jnp.transpose(x_ref[0].astype(jnp.bfloat16))

    def taps(src, d):
        return jnp.concatenate(
            [src[PAD - d:PAD - d + L, :],
             src[PAD:PAD + L, :],
             src[PAD + d:PAD + d + L, :]], axis=1)

    z = jnp.dot(taps(xs, 1), wi_ref[...],
                preferred_element_type=jnp.float32) + bi_ref[...]
    x0s[PAD:PAD + L, :] = z.astype(jnp.bfloat16)

    def gate(h):
        hb = h.astype(jnp.bfloat16)
        u = jnp.exp(jnp.minimum(-hb, jnp.bfloat16(30.0)))
        one = jnp.bfloat16(1.0)
        return (one - u) / (one + u * u)

    for li in range(4):
        d = 1 << li
        y1s = y1s_all[li]

        def block(src, idx):
            zz = jnp.dot(taps(src, d), wb_ref[idx],
                         preferred_element_type=jnp.float32) + bb_ref[idx]
            return gate(zz[:, :C]) + zz[:, C:].astype(jnp.bfloat16)

        y1s[PAD:PAD + L, :] = block(x0s, li)
        y2 = block(y1s, 4 + li)
        contrib = jnp.dot(y2, wf_ref[li], preferred_element_type=jnp.float32)
        if li == 0:
            acc = contrib + bf_ref[...]
        else:
            acc = acc + contrib
    out_ref[0] = jnp.transpose(acc)


def _fuse_block(cw, cb, rw, rb):
    w = jnp.zeros((3 * C, 2 * C), jnp.float32)
    w = w.at[:, :C].set(cw.reshape(3 * C, C))
    w = w.at[C:2 * C, C:].set(rw)
    b = jnp.concatenate([cb, rb], axis=-1)
    return w, b


def kernel(x, iw, ib, fw, fb,
           s0l0_cw, s0l0_cb, s0l0_rw, s0l0_rb,
           s0l1_cw, s0l1_cb, s0l1_rw, s0l1_rb,
           s0l2_cw, s0l2_cb, s0l2_rw, s0l2_rb,
           s0l3_cw, s0l3_cb, s0l3_rw, s0l3_rb,
           s1l0_cw, s1l0_cb, s1l0_rw, s1l0_rb, s1l0_sw, s1l0_sb,
           s1l1_cw, s1l1_cb, s1l1_rw, s1l1_rb, s1l1_sw, s1l1_sb,
           s1l2_cw, s1l2_cb, s1l2_rw, s1l2_rb, s1l2_sw, s1l2_sb,
           s1l3_cw, s1l3_cb, s1l3_rw, s1l3_rb, s1l3_sw, s1l3_sb):
    B, _, L = x.shape
    Lp = L + 2 * PAD

    blocks = [
        (s0l0_cw, s0l0_cb, s0l0_rw, s0l0_rb),
        (s0l1_cw, s0l1_cb, s0l1_rw, s0l1_rb),
        (s0l2_cw, s0l2_cb, s0l2_rw, s0l2_rb),
        (s0l3_cw, s0l3_cb, s0l3_rw, s0l3_rb),
        (s1l0_cw, s1l0_cb, s1l0_rw, s1l0_rb),
        (s1l1_cw, s1l1_cb, s1l1_rw, s1l1_rb),
        (s1l2_cw, s1l2_cb, s1l2_rw, s1l2_rb),
        (s1l3_cw, s1l3_cb, s1l3_rw, s1l3_rb),
    ]
    wb_list, bb_list = [], []
    for cw, cb, rw, rb in blocks:
        wt, bt = _fuse_block(cw, cb, rw, rb)
        wb_list.append(wt)
        bb_list.append(bt)
    wb = jnp.stack(wb_list).astype(jnp.bfloat16)
    bb = jnp.stack(bb_list)

    wi = iw.reshape(3 * C, C).astype(jnp.bfloat16)

    wf_chain = jnp.stack([sw @ fw for sw in
                          (s1l0_sw, s1l1_sw, s1l2_sw, s1l3_sw)]
                         ).astype(jnp.bfloat16)
    bsf = (s1l0_sb + s1l1_sb + s1l2_sb + s1l3_sb) @ fw + fb

    body = functools.partial(_wavenet_body, L=L)

    return pl.pallas_call(
        body,
        out_shape=jax.ShapeDtypeStruct((B, C, L), jnp.float32),
        grid=(B,),
        in_specs=[
            pl.BlockSpec((1, C, L), lambda b: (b, 0, 0)),
            pl.BlockSpec((3 * C, C), lambda b: (0, 0)),
            pl.BlockSpec((1, C), lambda b: (0, 0)),
            pl.BlockSpec((8, 3 * C, 2 * C), lambda b: (0, 0, 0)),
            pl.BlockSpec((8, 1, 2 * C), lambda b: (0, 0, 0)),
            pl.BlockSpec((4, C, C), lambda b: (0, 0, 0)),
            pl.BlockSpec((1, C), lambda b: (0, 0)),
        ],
        out_specs=pl.BlockSpec((1, C, L), lambda b: (b, 0, 0)),
        scratch_shapes=[
            pltpu.VMEM((Lp, C), jnp.bfloat16),
            pltpu.VMEM((Lp, C), jnp.bfloat16),
            pltpu.VMEM((Lp, C), jnp.bfloat16),
            pltpu.VMEM((Lp, C), jnp.bfloat16),
            pltpu.VMEM((Lp, C), jnp.bfloat16),
            pltpu.VMEM((Lp, C), jnp.bfloat16),
        ],
        compiler_params=pltpu.CompilerParams(
            dimension_semantics=("parallel",)),
    )(x, wi, ib, wb, bb, wf_chain, bsf)


# speedup vs baseline: 2.0825x; 1.2386x over previous
import functools

import jax
import jax.numpy as jnp
from jax.experimental import pallas as pl
from jax.experimental.pallas import tpu as pltpu

C = 128
PAD = 16


def _wavenet_body(x_ref, wi_ref, wb_ref, wf_ref, bf_ref, out_ref,
                  xs, x0s, y1a, y1b, y1c, y1d, *, L):
    y1s_all = (y1a, y1b, y1c, y1d)
    zh = jnp.zeros((PAD, C), jnp.bfloat16)
    for s in (xs, x0s) + y1s_all:
        s[0:PAD, :] = zh
        s[PAD + L:, :] = zh

    xs[PAD:PAD + L, :] = jnp.transpose(x_ref[0].astype(jnp.bfloat16))

    ones = jnp.ones((L, C), jnp.bfloat16)

    def taps(src, d):
        return jnp.concatenate(
            [src[PAD - d:PAD - d + L, :],
             src[PAD:PAD + L, :],
             src[PAD + d:PAD + d + L, :],
             ones], axis=1)

    z = jnp.dot(taps(xs, 1), wi_ref[...], preferred_element_type=jnp.float32)
    x0s[PAD:PAD + L, :] = z.astype(jnp.bfloat16)

    half = jnp.bfloat16(0.5)
    one = jnp.bfloat16(1.0)

    def gate(hb):
        return half * jnp.tanh(hb) * (one + jnp.tanh(hb * half))

    y2s = []
    for li in range(4):
        d = 1 << li
        y1s = y1s_all[li]

        def block(src, idx):
            zz = jnp.dot(taps(src, d), wb_ref[idx],
                         preferred_element_type=jnp.float32)
            zb = zz.astype(jnp.bfloat16)
            return gate(zb[:, :C]) + zb[:, C:]

        y1s[PAD:PAD + L, :] = block(x0s, li)
        y2s.append(block(y1s, 4 + li))

    ycat = jnp.concatenate(y2s, axis=1)
    acc = jnp.dot(ycat, wf_ref[...],
                  preferred_element_type=jnp.float32) + bf_ref[...]
    out_ref[0] = jnp.transpose(acc)


def _fuse_block(cw, cb, rw, rb):
    w = jnp.zeros((4 * C, 2 * C), jnp.float32)
    w = w.at[:3 * C, :C].set(cw.reshape(3 * C, C))
    w = w.at[C:2 * C, C:].set(rw)
    w = w.at[3 * C, :C].set(cb[0])
    w = w.at[3 * C, C:].set(rb[0])
    return w


def kernel(x, iw, ib, fw, fb,
           s0l0_cw, s0l0_cb, s0l0_rw, s0l0_rb,
           s0l1_cw, s0l1_cb, s0l1_rw, s0l1_rb,
           s0l2_cw, s0l2_cb, s0l2_rw, s0l2_rb,
           s0l3_cw, s0l3_cb, s0l3_rw, s0l3_rb,
           s1l0_cw, s1l0_cb, s1l0_rw, s1l0_rb, s1l0_sw, s1l0_sb,
           s1l1_cw, s1l1_cb, s1l1_rw, s1l1_rb, s1l1_sw, s1l1_sb,
           s1l2_cw, s1l2_cb, s1l2_rw, s1l2_rb, s1l2_sw, s1l2_sb,
           s1l3_cw, s1l3_cb, s1l3_rw, s1l3_rb, s1l3_sw, s1l3_sb):
    B, _, L = x.shape
    Lp = L + 2 * PAD

    blocks = [
        (s0l0_cw, s0l0_cb, s0l0_rw, s0l0_rb),
        (s0l1_cw, s0l1_cb, s0l1_rw, s0l1_rb),
        (s0l2_cw, s0l2_cb, s0l2_rw, s0l2_rb),
        (s0l3_cw, s0l3_cb, s0l3_rw, s0l3_rb),
        (s1l0_cw, s1l0_cb, s1l0_rw, s1l0_rb),
        (s1l1_cw, s1l1_cb, s1l1_rw, s1l1_rb),
        (s1l2_cw, s1l2_cb, s1l2_rw, s1l2_rb),
        (s1l3_cw, s1l3_cb, s1l3_rw, s1l3_rb),
    ]
    wb = jnp.stack([_fuse_block(*blk) for blk in blocks]
                   ).astype(jnp.bfloat16)

    wi = jnp.zeros((4 * C, C), jnp.float32)
    wi = wi.at[:3 * C, :].set(iw.reshape(3 * C, C))
    wi = wi.at[3 * C, :].set(ib[0])
    wi = wi.astype(jnp.bfloat16)

    wf_chain = jnp.concatenate([sw @ fw for sw in
                                (s1l0_sw, s1l1_sw, s1l2_sw, s1l3_sw)],
                               axis=0).astype(jnp.bfloat16)
    bsf = (s1l0_sb + s1l1_sb + s1l2_sb + s1l3_sb) @ fw + fb

    body = functools.partial(_wavenet_body, L=L)

    return pl.pallas_call(
        body,
        out_shape=jax.ShapeDtypeStruct((B, C, L), jnp.float32),
        grid=(B,),
        in_specs=[
            pl.BlockSpec((1, C, L), lambda b: (b, 0, 0)),
            pl.BlockSpec((4 * C, C), lambda b: (0, 0)),
            pl.BlockSpec((8, 4 * C, 2 * C), lambda b: (0, 0, 0)),
            pl.BlockSpec((4 * C, C), lambda b: (0, 0)),
            pl.BlockSpec((1, C), lambda b: (0, 0)),
        ],
        out_specs=pl.BlockSpec((1, C, L), lambda b: (b, 0, 0)),
        scratch_shapes=[
            pltpu.VMEM((Lp, C), jnp.bfloat16),
            pltpu.VMEM((Lp, C), jnp.bfloat16),
            pltpu.VMEM((Lp, C), jnp.bfloat16),
            pltpu.VMEM((Lp, C), jnp.bfloat16),
            pltpu.VMEM((Lp, C), jnp.bfloat16),
            pltpu.VMEM((Lp, C), jnp.bfloat16),
        ],
        compiler_params=pltpu.CompilerParams(
            dimension_semantics=("parallel",)),
    )(x, wi, wb, wf_chain, bsf)
